# Initial kernel scaffold; baseline (speedup 1.0000x reference)
import jax
import jax.numpy as jnp
from jax.experimental import pallas as pl


def kernel(x, W_enc, b_enc, W_dec, b_dec):
    raise NotImplementedError("write your pallas kernel here")


# speedup vs baseline: 7.4272x; 7.4272x over previous
import functools

import jax
import jax.numpy as jnp
from jax.experimental import pallas as pl
from jax.experimental.pallas import tpu as pltpu

K = 64


def _encode_topk_body(x_ref, wenc_ref, benc_ref, bdec_ref, h_ref, *, n_cblks, bc):
    c = pl.program_id(1)
    xc = x_ref[...] - bdec_ref[...]
    acts = jax.lax.dot_general(
        xc, wenc_ref[...], (((1,), (1,)), ((), ())),
        preferred_element_type=jnp.float32)
    acts = acts + benc_ref[...]
    bits = jax.lax.bitcast_convert_type(acts, jnp.int32)
    s = jnp.where(bits < 0, bits ^ jnp.int32(0x7FFFFFFF), bits)
    h_ref[:, pl.ds(c * bc, bc)] = jax.lax.bitcast_convert_type(s, jnp.float32)

    @pl.when(c == n_cblks - 1)
    def _():
        sfull = jax.lax.bitcast_convert_type(h_ref[...], jnp.int32)
        cnt_pos = jnp.sum((sfull >= 0).astype(jnp.int32), axis=1, keepdims=True)
        t0 = jnp.where(cnt_pos >= K, 0, -(2 ** 31)).astype(jnp.int32)

        def body(i, t):
            bit = jnp.int32(1) << (jnp.int32(30) - i)
            t_try = t | bit
            cnt = jnp.sum((sfull >= t_try).astype(jnp.int32), axis=1,
                          keepdims=True)
            return jnp.where(cnt >= K, t_try, t)

        t = jax.lax.fori_loop(0, 31, body, t0, unroll=True)
        thr = jnp.maximum(t, 0)
        h_ref[...] = jnp.where(
            sfull >= thr, jax.lax.bitcast_convert_type(sfull, jnp.float32), 0.0)


def _decode_loss_body(h_ref, wdec_ref, bdec_ref, x_ref, xh_ref, loss_ref, *,
                      n_kblks):
    k = pl.program_id(1)

    @pl.when(k == 0)
    def _():
        xh_ref[...] = jnp.broadcast_to(bdec_ref[...], xh_ref.shape)

    xh_ref[...] += jax.lax.dot_general(
        h_ref[...], wdec_ref[...], (((1,), (1,)), ((), ())),
        preferred_element_type=jnp.float32)

    @pl.when(k == n_kblks - 1)
    def _():
        d = xh_ref[...] - x_ref[...]
        loss_ref[...] = jnp.broadcast_to(
            jnp.sum(d * d).reshape(1, 1, 1), loss_ref.shape)


def _pick(n, pref):
    for b in (pref, pref // 2, pref // 4):
        if b and n % b == 0:
            return b
    return n


@jax.jit
def kernel(x, W_enc, b_enc, W_dec, b_dec):
    n, dm = x.shape
    s = W_enc.shape[0]
    br = _pick(n, 256)
    bc = _pick(s, 512)
    n_rblks, n_cblks = n // br, s // bc

    benc2 = b_enc.reshape(1, s)
    bdec2 = b_dec.reshape(1, dm)

    h = pl.pallas_call(
        functools.partial(_encode_topk_body, n_cblks=n_cblks, bc=bc),
        grid=(n_rblks, n_cblks),
        in_specs=[
            pl.BlockSpec((br, dm), lambda r, c: (r, 0)),
            pl.BlockSpec((bc, dm), lambda r, c: (c, 0)),
            pl.BlockSpec((1, bc), lambda r, c: (0, c)),
            pl.BlockSpec((1, dm), lambda r, c: (0, 0)),
        ],
        out_specs=pl.BlockSpec((br, s), lambda r, c: (r, 0)),
        out_shape=jax.ShapeDtypeStruct((n, s), jnp.float32),
        compiler_params=pltpu.CompilerParams(
            dimension_semantics=("arbitrary", "arbitrary")),
    )(x, W_enc, benc2, bdec2)

    br2 = _pick(n, 1024)
    bk = _pick(s, 512)
    n_r2, n_kblks = n // br2, s // bk

    x_hat, loss_parts = pl.pallas_call(
        functools.partial(_decode_loss_body, n_kblks=n_kblks),
        grid=(n_r2, n_kblks),
        in_specs=[
            pl.BlockSpec((br2, bk), lambda r, k: (r, k)),
            pl.BlockSpec((dm, bk), lambda r, k: (0, k)),
            pl.BlockSpec((1, dm), lambda r, k: (0, 0)),
            pl.BlockSpec((br2, dm), lambda r, k: (r, 0)),
        ],
        out_specs=[
            pl.BlockSpec((br2, dm), lambda r, k: (r, 0)),
            pl.BlockSpec((1, 1, 128), lambda r, k: (r, 0, 0)),
        ],
        out_shape=[
            jax.ShapeDtypeStruct((n, dm), jnp.float32),
            jax.ShapeDtypeStruct((n_r2, 1, 128), jnp.float32),
        ],
        compiler_params=pltpu.CompilerParams(
            dimension_semantics=("arbitrary", "arbitrary")),
    )(h, W_dec, bdec2, x)

    loss = jnp.sum(loss_parts[:, 0, 0]) / n
    return (x_hat, h, loss)


# speedup vs baseline: 8.4377x; 1.1360x over previous
import functools

import jax
import jax.numpy as jnp
from jax.experimental import pallas as pl
from jax.experimental.pallas import tpu as pltpu

K = 64


def _encode_topk_body(x_ref, wenc_ref, benc_ref, bdec_ref, h_ref, s_scr,
                      t_scr, *, n_cblks, bc):
    c = pl.program_id(1)

    @pl.when(c < n_cblks)
    def _compute():
        xc = x_ref[...] - bdec_ref[...]
        acts = jax.lax.dot_general(
            xc, wenc_ref[...], (((1,), (1,)), ((), ())),
            preferred_element_type=jnp.float32)
        acts = acts + benc_ref[...]
        bits = jax.lax.bitcast_convert_type(acts, jnp.int32)
        s_scr[:, pl.ds(c * bc, bc)] = jnp.where(
            bits < 0, bits ^ jnp.int32(0x7FFFFFFF), bits)

    @pl.when(c == n_cblks - 1)
    def _search():
        sfull = s_scr[...]
        cnt_pos = jnp.sum((sfull >= 0).astype(jnp.int32), axis=1, keepdims=True)
        t0 = jnp.where(cnt_pos >= K, 0, -(2 ** 31)).astype(jnp.int32)

        def body(i, t):
            bit = jnp.int32(1) << (jnp.int32(30) - i)
            t_try = t | bit
            cnt = jnp.sum((sfull >= t_try).astype(jnp.int32), axis=1,
                          keepdims=True)
            return jnp.where(cnt >= K, t_try, t)

        t = jax.lax.fori_loop(0, 31, body, t0, unroll=True)
        t_scr[...] = jnp.broadcast_to(jnp.maximum(t, 0), t_scr.shape)

    @pl.when(c >= n_cblks)
    def _writeback():
        j = c - n_cblks
        s_chunk = s_scr[:, pl.ds(j * bc, bc)]
        thr = t_scr[:, 0:1]
        h_ref[...] = jnp.where(
            s_chunk >= thr, jax.lax.bitcast_convert_type(s_chunk, jnp.float32),
            0.0)


def _decode_loss_body(h_ref, wdec_ref, bdec_ref, x_ref, xh_ref, loss_ref, *,
                      n_kblks):
    k = pl.program_id(1)

    @pl.when(k == 0)
    def _():
        xh_ref[...] = jnp.broadcast_to(bdec_ref[...], xh_ref.shape)

    xh_ref[...] += jax.lax.dot_general(
        h_ref[...].astype(jnp.bfloat16), wdec_ref[...],
        (((1,), (1,)), ((), ())),
        preferred_element_type=jnp.float32)

    @pl.when(k == n_kblks - 1)
    def _():
        d = xh_ref[...] - x_ref[...]
        loss_ref[...] = jnp.broadcast_to(
            jnp.sum(d * d).reshape(1, 1, 1), loss_ref.shape)


def _pick(n, pref):
    for b in (pref, pref // 2, pref // 4):
        if b and n % b == 0:
            return b
    return n


@jax.jit
def kernel(x, W_enc, b_enc, W_dec, b_dec):
    n, dm = x.shape
    s = W_enc.shape[0]
    br = _pick(n, 512)
    bc = _pick(s, 512)
    n_rblks, n_cblks = n // br, s // bc

    benc2 = b_enc.reshape(1, s)
    bdec2 = b_dec.reshape(1, dm)

    h = pl.pallas_call(
        functools.partial(_encode_topk_body, n_cblks=n_cblks, bc=bc),
        grid=(n_rblks, 2 * n_cblks),
        in_specs=[
            pl.BlockSpec((br, dm), lambda r, c: (r, 0)),
            pl.BlockSpec((bc, dm),
                         lambda r, c: (jnp.minimum(c, n_cblks - 1), 0)),
            pl.BlockSpec((1, bc),
                         lambda r, c: (0, jnp.minimum(c, n_cblks - 1))),
            pl.BlockSpec((1, dm), lambda r, c: (0, 0)),
        ],
        out_specs=pl.BlockSpec(
            (br, bc), lambda r, c: (r, jnp.maximum(c - n_cblks, 0))),
        out_shape=jax.ShapeDtypeStruct((n, s), jnp.float32),
        scratch_shapes=[
            pltpu.VMEM((br, s), jnp.int32),
            pltpu.VMEM((br, 128), jnp.int32),
        ],
        compiler_params=pltpu.CompilerParams(
            dimension_semantics=("arbitrary", "arbitrary")),
    )(x, W_enc, benc2, bdec2)

    br2 = _pick(n, 1024)
    bk = _pick(s, 512)
    n_r2, n_kblks = n // br2, s // bk
    wdec_bf = W_dec.astype(jnp.bfloat16)

    x_hat, loss_parts = pl.pallas_call(
        functools.partial(_decode_loss_body, n_kblks=n_kblks),
        grid=(n_r2, n_kblks),
        in_specs=[
            pl.BlockSpec((br2, bk), lambda r, k: (r, k)),
            pl.BlockSpec((dm, bk), lambda r, k: (0, k)),
            pl.BlockSpec((1, dm), lambda r, k: (0, 0)),
            pl.BlockSpec((br2, dm), lambda r, k: (r, 0)),
        ],
        out_specs=[
            pl.BlockSpec((br2, dm), lambda r, k: (r, 0)),
            pl.BlockSpec((1, 1, 128), lambda r, k: (r, 0, 0)),
        ],
        out_shape=[
            jax.ShapeDtypeStruct((n, dm), jnp.float32),
            jax.ShapeDtypeStruct((n_r2, 1, 128), jnp.float32),
        ],
        compiler_params=pltpu.CompilerParams(
            dimension_semantics=("arbitrary", "arbitrary")),
    )(h, wdec_bf, bdec2, x)

    loss = jnp.sum(loss_parts[:, 0, 0]) / n
    return (x_hat, h, loss)
